# baseline (speedup 1.0000x reference)
import functools

import jax
import jax.numpy as jnp
from jax import lax
from jax.experimental import pallas as pl
from jax.experimental.pallas import tpu as pltpu
from jax.experimental.pallas import tpu_sc as plsc

N_DST = 10000
D = 128
CHUNK = 80
NC, NS = 2, 16
NW = NC * NS
RPT = 624
ZBLK = 16


def _sc_partials(src_emb, idx3d, n_edges):
    E = n_edges
    epw = E // NW
    cpw = epw // CHUNK
    assert epw * NW == E and cpw * CHUNK == epw
    mesh = plsc.VectorSubcoreMesh(core_axis_name="c", subcore_axis_name="s")

    @functools.partial(
        pl.kernel,
        out_type=jax.ShapeDtypeStruct((NC, N_DST, D), jnp.float32),
        mesh=mesh,
        scratch_types=[
            pltpu.VMEM_SHARED((N_DST, D), jnp.float32),
            pltpu.VMEM((cpw, CHUNK), jnp.int32),
            pltpu.VMEM((2, CHUNK, D), jnp.float32),
            pltpu.VMEM((ZBLK, D), jnp.float32),
            pltpu.SemaphoreType.DMA,
            pltpu.SemaphoreType.DMA,
        ],
    )
    def k(src_hbm, idx_hbm, out_hbm, acc, idx_v, rows_v, zero_v, sem0, sem1):
        c = lax.axis_index("c")
        s = lax.axis_index("s")
        wid = s * NC + c
        r0 = s * RPT
        ebase = wid * epw
        sems = (sem0, sem1)

        def gstart(j, b):
            pltpu.async_copy(src_hbm.at[pl.ds(ebase + j * CHUNK, CHUNK)],
                             rows_v.at[b], sems[b])

        def gwait(j, b):
            pltpu.make_async_copy(src_hbm.at[pl.ds(ebase + j * CHUNK, CHUNK)],
                                  rows_v.at[b], sems[b]).wait()

        def scat(j, b):
            pltpu.sync_copy(rows_v.at[b], acc.at[idx_v.at[j]], add=True)

        pltpu.sync_copy(idx_hbm.at[wid], idx_v)
        gstart(0, 0)
        gstart(1, 1)

        @pl.when(c == 0)
        def _():
            pltpu.sync_copy(src_hbm.at[pl.ds(E + r0, RPT)], acc.at[pl.ds(r0, RPT)])

            @pl.when(s == NS - 1)
            def _():
                pltpu.sync_copy(src_hbm.at[pl.ds(E + NS * RPT, N_DST - NS * RPT)],
                                acc.at[pl.ds(NS * RPT, N_DST - NS * RPT)])

        @pl.when(c == 1)
        def _():
            def zrow(r, carry):
                for col in range(D // 16):
                    zero_v[r, pl.ds(col * 16, 16)] = jnp.zeros((16,), jnp.float32)
                return carry
            lax.fori_loop(0, ZBLK, zrow, 0)
            for kk in range(RPT // ZBLK):
                pltpu.sync_copy(zero_v, acc.at[pl.ds(r0 + kk * ZBLK, ZBLK)])

            @pl.when(s == NS - 1)
            def _():
                pltpu.sync_copy(zero_v.at[pl.ds(0, N_DST - NS * RPT)],
                                acc.at[pl.ds(NS * RPT, N_DST - NS * RPT)])

        plsc.subcore_barrier()

        npairs = (cpw - 2) // 2

        def body(g, carry):
            for b in range(2):
                j = g * 2 + b
                gwait(j, b)
                scat(j, b)
                gstart(j + 2, b)
            return carry
        lax.fori_loop(0, npairs, body, 0)
        for j in range(2 * npairs, cpw):
            b = j % 2
            gwait(j, b)
            scat(j, b)
            if j + 2 < cpw:
                gstart(j + 2, b)

        plsc.subcore_barrier()
        pltpu.sync_copy(acc.at[pl.ds(r0, RPT)], out_hbm.at[c, pl.ds(r0, RPT)])

        @pl.when(s == NS - 1)
        def _():
            pltpu.sync_copy(acc.at[pl.ds(NS * RPT, N_DST - NS * RPT)],
                            out_hbm.at[c, pl.ds(NS * RPT, N_DST - NS * RPT)])

    return k(src_emb, idx3d)


def _combine(partials):
    blk = 2000

    def add_k(p_ref, o_ref):
        o_ref[...] = p_ref[0] + p_ref[1]

    return pl.pallas_call(
        add_k,
        grid=(N_DST // blk,),
        in_specs=[pl.BlockSpec((NC, blk, D), lambda i: (0, i, 0))],
        out_specs=pl.BlockSpec((blk, D), lambda i: (i, 0)),
        out_shape=jax.ShapeDtypeStruct((N_DST, D), jnp.float32),
    )(partials)


def kernel(src_emb, src_emb_in, dst_ids):
    del src_emb_in
    E = dst_ids.shape[0]
    epw = E // NW
    idx3d = dst_ids.astype(jnp.int32).reshape(NW, epw // CHUNK, CHUNK)
    partials = _sc_partials(src_emb, idx3d, E)
    return _combine(partials)


# speedup vs baseline: 1.1852x; 1.1852x over previous
import functools

import jax
import jax.numpy as jnp
from jax import lax
from jax.experimental import pallas as pl
from jax.experimental.pallas import tpu as pltpu
from jax.experimental.pallas import tpu_sc as plsc

N_DST = 10000
D = 128
CHUNK = 80
NC, NS = 2, 16
NW = NC * NS
RPT = 624
ZBLK = 16
NBUF = 3


def _sc_partials(src_emb, idx3d, n_edges):
    E = n_edges
    epw = E // NW
    cpw = epw // CHUNK
    assert epw * NW == E and cpw * CHUNK == epw
    mesh = plsc.VectorSubcoreMesh(core_axis_name="c", subcore_axis_name="s")

    @functools.partial(
        pl.kernel,
        out_type=jax.ShapeDtypeStruct((NC, N_DST, D), jnp.float32),
        mesh=mesh,
        scratch_types=[
            pltpu.VMEM_SHARED((N_DST, D), jnp.float32),
            pltpu.VMEM((cpw, CHUNK), jnp.int32),
            pltpu.VMEM((NBUF, CHUNK, D), jnp.float32),
            pltpu.VMEM((ZBLK, D), jnp.float32),
        ] + [pltpu.SemaphoreType.DMA] * NBUF,
    )
    def k(src_hbm, idx_hbm, out_hbm, acc, idx_v, rows_v, zero_v, *sems):
        c = lax.axis_index("c")
        s = lax.axis_index("s")
        wid = s * NC + c
        r0 = s * RPT
        ebase = wid * epw

        def gstart(j, b):
            pltpu.async_copy(src_hbm.at[pl.ds(ebase + j * CHUNK, CHUNK)],
                             rows_v.at[b], sems[b])

        def gwait(j, b):
            pltpu.make_async_copy(src_hbm.at[pl.ds(ebase + j * CHUNK, CHUNK)],
                                  rows_v.at[b], sems[b]).wait()

        def scat(j, b):
            pltpu.sync_copy(rows_v.at[b], acc.at[idx_v.at[j]], add=True)

        pltpu.sync_copy(idx_hbm.at[wid], idx_v)
        for b in range(NBUF):
            gstart(b, b)

        @pl.when(c == 0)
        def _():
            pltpu.sync_copy(src_hbm.at[pl.ds(E + r0, RPT)], acc.at[pl.ds(r0, RPT)])

            @pl.when(s == NS - 1)
            def _():
                pltpu.sync_copy(src_hbm.at[pl.ds(E + NS * RPT, N_DST - NS * RPT)],
                                acc.at[pl.ds(NS * RPT, N_DST - NS * RPT)])

        @pl.when(c == 1)
        def _():
            def zrow(r, carry):
                for col in range(D // 16):
                    zero_v[r, pl.ds(col * 16, 16)] = jnp.zeros((16,), jnp.float32)
                return carry
            lax.fori_loop(0, ZBLK, zrow, 0)
            for kk in range(RPT // ZBLK):
                pltpu.sync_copy(zero_v, acc.at[pl.ds(r0 + kk * ZBLK, ZBLK)])

            @pl.when(s == NS - 1)
            def _():
                pltpu.sync_copy(zero_v.at[pl.ds(0, N_DST - NS * RPT)],
                                acc.at[pl.ds(NS * RPT, N_DST - NS * RPT)])

        plsc.subcore_barrier()

        ngroups = (cpw - NBUF) // NBUF

        def body(g, carry):
            for b in range(NBUF):
                j = g * NBUF + b
                gwait(j, b)
                scat(j, b)
                gstart(j + NBUF, b)
            return carry
        lax.fori_loop(0, ngroups, body, 0)
        for j in range(NBUF * ngroups, cpw):
            b = j % NBUF
            gwait(j, b)
            scat(j, b)
            if j + NBUF < cpw:
                gstart(j + NBUF, b)

        plsc.subcore_barrier()
        pltpu.sync_copy(acc.at[pl.ds(r0, RPT)], out_hbm.at[c, pl.ds(r0, RPT)])

        @pl.when(s == NS - 1)
        def _():
            pltpu.sync_copy(acc.at[pl.ds(NS * RPT, N_DST - NS * RPT)],
                            out_hbm.at[c, pl.ds(NS * RPT, N_DST - NS * RPT)])

    return k(src_emb, idx3d)


def _combine(partials):
    blk = 2000

    def add_k(p_ref, o_ref):
        o_ref[...] = p_ref[0] + p_ref[1]

    return pl.pallas_call(
        add_k,
        grid=(N_DST // blk,),
        in_specs=[pl.BlockSpec((NC, blk, D), lambda i: (0, i, 0))],
        out_specs=pl.BlockSpec((blk, D), lambda i: (i, 0)),
        out_shape=jax.ShapeDtypeStruct((N_DST, D), jnp.float32),
    )(partials)


def kernel(src_emb, src_emb_in, dst_ids):
    del src_emb_in
    E = dst_ids.shape[0]
    epw = E // NW
    idx3d = dst_ids.astype(jnp.int32).reshape(NW, epw // CHUNK, CHUNK)
    partials = _sc_partials(src_emb, idx3d, E)
    return _combine(partials)


# speedup vs baseline: 1.2096x; 1.0206x over previous
import functools

import jax
import jax.numpy as jnp
from jax import lax
from jax.experimental import pallas as pl
from jax.experimental.pallas import tpu as pltpu
from jax.experimental.pallas import tpu_sc as plsc

N_DST = 10000
D = 128
CHUNK = 80
NC, NS = 2, 16
NW = NC * NS
RPT = 624
ZBLK = 24
NBUF = 3


def _sc_partials(src_emb, idx3d, n_edges):
    E = n_edges
    epw = E // NW
    cpw = epw // CHUNK
    assert epw * NW == E and cpw * CHUNK == epw
    mesh = plsc.VectorSubcoreMesh(core_axis_name="c", subcore_axis_name="s")

    @functools.partial(
        pl.kernel,
        out_type=jax.ShapeDtypeStruct((NC, N_DST, D), jnp.float32),
        mesh=mesh,
        scratch_types=[
            pltpu.VMEM_SHARED((N_DST, D), jnp.float32),
            pltpu.VMEM((cpw, CHUNK), jnp.int32),
            pltpu.VMEM((NBUF, CHUNK, D), jnp.float32),
            pltpu.VMEM((ZBLK, D), jnp.float32),
        ] + [pltpu.SemaphoreType.DMA] * (NBUF + 1),
    )
    def k(src_hbm, idx_hbm, out_hbm, acc, idx_v, rows_v, zero_v, zsem, *sems):
        c = lax.axis_index("c")
        s = lax.axis_index("s")
        wid = s * NC + c
        r0 = s * RPT
        ebase = wid * epw

        def gstart(j, b):
            pltpu.async_copy(src_hbm.at[pl.ds(ebase + j * CHUNK, CHUNK)],
                             rows_v.at[b], sems[b])

        def gwait(j, b):
            pltpu.make_async_copy(src_hbm.at[pl.ds(ebase + j * CHUNK, CHUNK)],
                                  rows_v.at[b], sems[b]).wait()

        def scat(j, b):
            pltpu.sync_copy(rows_v.at[b], acc.at[idx_v.at[j]], add=True)

        pltpu.sync_copy(idx_hbm.at[wid], idx_v)
        for b in range(NBUF):
            gstart(b, b)

        def zrow(r, carry):
            for col in range(D // 16):
                zero_v[r, pl.ds(col * 16, 16)] = jnp.zeros((16,), jnp.float32)
            return carry
        lax.fori_loop(0, ZBLK, zrow, 0)
        for kk in range(RPT // ZBLK):
            pltpu.async_copy(zero_v, acc.at[pl.ds(r0 + kk * ZBLK, ZBLK)], zsem)

        @pl.when(s == NS - 1)
        def _():
            pltpu.async_copy(zero_v.at[pl.ds(0, N_DST - NS * RPT)],
                             acc.at[pl.ds(NS * RPT, N_DST - NS * RPT)], zsem)
        for kk in range(RPT // ZBLK):
            pltpu.make_async_copy(zero_v, acc.at[pl.ds(r0 + kk * ZBLK, ZBLK)],
                                  zsem).wait()

        @pl.when(s == NS - 1)
        def _():
            pltpu.make_async_copy(zero_v.at[pl.ds(0, N_DST - NS * RPT)],
                                  acc.at[pl.ds(NS * RPT, N_DST - NS * RPT)],
                                  zsem).wait()

        plsc.subcore_barrier()

        ngroups = (cpw - NBUF) // NBUF

        def body(g, carry):
            for b in range(NBUF):
                j = g * NBUF + b
                gwait(j, b)
                scat(j, b)
                gstart(j + NBUF, b)
            return carry
        lax.fori_loop(0, ngroups, body, 0)
        for j in range(NBUF * ngroups, cpw):
            b = j % NBUF
            gwait(j, b)
            scat(j, b)
            if j + NBUF < cpw:
                gstart(j + NBUF, b)

        plsc.subcore_barrier()
        pltpu.sync_copy(acc.at[pl.ds(r0, RPT)], out_hbm.at[c, pl.ds(r0, RPT)])

        @pl.when(s == NS - 1)
        def _():
            pltpu.sync_copy(acc.at[pl.ds(NS * RPT, N_DST - NS * RPT)],
                            out_hbm.at[c, pl.ds(NS * RPT, N_DST - NS * RPT)])

    return k(src_emb, idx3d)


def _combine(partials, src_emb, n_edges):
    blk = 2000
    ofs = n_edges // blk
    assert ofs * blk == n_edges

    def add_k(p_ref, self_ref, o_ref):
        o_ref[...] = p_ref[0] + p_ref[1] + self_ref[...]

    return pl.pallas_call(
        add_k,
        grid=(N_DST // blk,),
        in_specs=[pl.BlockSpec((NC, blk, D), lambda i: (0, i, 0)),
                  pl.BlockSpec((blk, D), lambda i: (ofs + i, 0))],
        out_specs=pl.BlockSpec((blk, D), lambda i: (i, 0)),
        out_shape=jax.ShapeDtypeStruct((N_DST, D), jnp.float32),
    )(partials, src_emb)


def kernel(src_emb, src_emb_in, dst_ids):
    del src_emb_in
    E = dst_ids.shape[0]
    epw = E // NW
    idx3d = dst_ids.astype(jnp.int32).reshape(NW, epw // CHUNK, CHUNK)
    partials = _sc_partials(src_emb, idx3d, E)
    return _combine(partials, src_emb, E)


# speedup vs baseline: 1.2275x; 1.0148x over previous
import functools

import jax
import jax.numpy as jnp
from jax import lax
from jax.experimental import pallas as pl
from jax.experimental.pallas import tpu as pltpu
from jax.experimental.pallas import tpu_sc as plsc

N_DST = 10000
D = 128
CHUNK = 80
NC, NS = 2, 16
NW = NC * NS
RPT = 624
ZBLK = 8
NBUF = 4
IQ = 32


def _sc_partials(src_emb, idx3d, n_edges):
    E = n_edges
    epw = E // NW
    cpw = epw // CHUNK
    nq = idx3d.shape[1] // IQ
    assert epw * NW == E and cpw * CHUNK == epw and nq * IQ >= cpw
    mesh = plsc.VectorSubcoreMesh(core_axis_name="c", subcore_axis_name="s")

    @functools.partial(
        pl.kernel,
        out_type=jax.ShapeDtypeStruct((NC, N_DST, D), jnp.float32),
        mesh=mesh,
        scratch_types=[
            pltpu.VMEM_SHARED((N_DST, D), jnp.float32),
            pltpu.VMEM((2, IQ, CHUNK), jnp.int32),
            pltpu.VMEM((NBUF, CHUNK, D), jnp.float32),
            pltpu.VMEM((ZBLK, D), jnp.float32),
        ] + [pltpu.SemaphoreType.DMA] * (NBUF + 3),
    )
    def k(src_hbm, idx_hbm, out_hbm, acc, idx_q, rows_v, zero_v,
          zsem, isem0, isem1, *sems):
        c = lax.axis_index("c")
        s = lax.axis_index("s")
        wid = s * NC + c
        r0 = s * RPT
        ebase = wid * epw
        isems = (isem0, isem1)

        def gstart(j, b):
            pltpu.async_copy(src_hbm.at[pl.ds(ebase + j * CHUNK, CHUNK)],
                             rows_v.at[b], sems[b])

        def gwait(j, b):
            pltpu.make_async_copy(src_hbm.at[pl.ds(ebase + j * CHUNK, CHUNK)],
                                  rows_v.at[b], sems[b]).wait()

        def istart(q):
            pltpu.async_copy(idx_hbm.at[wid, pl.ds(q * IQ, IQ)],
                             idx_q.at[q % 2], isems[q % 2])

        def iwait(q):
            pltpu.make_async_copy(idx_hbm.at[wid, pl.ds(q * IQ, IQ)],
                                  idx_q.at[q % 2], isems[q % 2]).wait()

        def scat(j, b, q):
            pltpu.sync_copy(rows_v.at[b],
                            acc.at[idx_q.at[q % 2, j - q * IQ]], add=True)

        istart(0)
        for b in range(NBUF):
            gstart(b, b)

        def zrow(r, carry):
            for col in range(D // 16):
                zero_v[r, pl.ds(col * 16, 16)] = jnp.zeros((16,), jnp.float32)
            return carry
        lax.fori_loop(0, ZBLK, zrow, 0)
        for kk in range(RPT // ZBLK):
            pltpu.async_copy(zero_v, acc.at[pl.ds(r0 + kk * ZBLK, ZBLK)], zsem)

        nxtra = (N_DST - NS * RPT) // ZBLK

        @pl.when(s == NS - 1)
        def _():
            for kk in range(nxtra):
                pltpu.async_copy(
                    zero_v, acc.at[pl.ds(NS * RPT + kk * ZBLK, ZBLK)], zsem)
        for kk in range(RPT // ZBLK):
            pltpu.make_async_copy(zero_v, acc.at[pl.ds(r0 + kk * ZBLK, ZBLK)],
                                  zsem).wait()

        @pl.when(s == NS - 1)
        def _():
            for kk in range(nxtra):
                pltpu.make_async_copy(
                    zero_v, acc.at[pl.ds(NS * RPT + kk * ZBLK, ZBLK)],
                    zsem).wait()

        iwait(0)
        plsc.subcore_barrier()

        gpq = IQ // NBUF
        nfull = (cpw - 1) // NBUF
        for q in range(nq):
            if q >= 1:
                iwait(q)
            if q + 1 < nq:
                istart(q + 1)

            def body(g, carry, q=q):
                for b in range(NBUF):
                    j = g * NBUF + b
                    gwait(j, b)
                    scat(j, b, q)
                    if q + 1 < nq:
                        gstart(j + NBUF, b)
                    else:
                        @pl.when(j + NBUF < cpw)
                        def _():
                            gstart(j + NBUF, b)
                return carry
            lax.fori_loop(q * gpq, min((q + 1) * gpq, nfull), body, 0)
        for j in range(NBUF * nfull, cpw):
            gwait(j, j % NBUF)
            scat(j, j % NBUF, j // IQ)

        plsc.subcore_barrier()
        pltpu.sync_copy(acc.at[pl.ds(r0, RPT)], out_hbm.at[c, pl.ds(r0, RPT)])

        @pl.when(s == NS - 1)
        def _():
            pltpu.sync_copy(acc.at[pl.ds(NS * RPT, N_DST - NS * RPT)],
                            out_hbm.at[c, pl.ds(NS * RPT, N_DST - NS * RPT)])

    return k(src_emb, idx3d)


def _combine(partials, src_emb, n_edges):
    blk = 2000
    ofs = n_edges // blk
    assert ofs * blk == n_edges

    def add_k(p_ref, self_ref, o_ref):
        o_ref[...] = p_ref[0] + p_ref[1] + self_ref[...]

    return pl.pallas_call(
        add_k,
        grid=(N_DST // blk,),
        in_specs=[pl.BlockSpec((NC, blk, D), lambda i: (0, i, 0)),
                  pl.BlockSpec((blk, D), lambda i: (ofs + i, 0))],
        out_specs=pl.BlockSpec((blk, D), lambda i: (i, 0)),
        out_shape=jax.ShapeDtypeStruct((N_DST, D), jnp.float32),
    )(partials, src_emb)


def kernel(src_emb, src_emb_in, dst_ids):
    del src_emb_in
    E = dst_ids.shape[0]
    epw = E // NW
    cpw = epw // CHUNK
    nq = -(-cpw // IQ)
    idx3d = dst_ids.astype(jnp.int32).reshape(NW, cpw, CHUNK)
    idx3d = jnp.pad(idx3d, ((0, 0), (0, nq * IQ - cpw), (0, 0)))
    partials = _sc_partials(src_emb, idx3d, E)
    return _combine(partials, src_emb, E)
